# Initial kernel scaffold; baseline (speedup 1.0000x reference)
import jax
import jax.numpy as jnp
from jax.experimental import pallas as pl


def kernel(rois, pooled_features, fc1_w, fc1_b, fc2_w, fc2_b):
    raise NotImplementedError("write your pallas kernel here")


# speedup vs baseline: 2.4035x; 2.4035x over previous
import functools

import jax
import jax.numpy as jnp
from jax import lax
from jax.experimental import pallas as pl
from jax.experimental.pallas import tpu as pltpu
from jax.experimental.pallas import tpu_sc as plsc

B, N = 16, 1024
BN = B * N
K = 32
R2 = 1.0
D = 64
PAD = BN

NC, NS, L = 2, 16, 16
NW = NC * NS
NPW = BN // NW
CHUNK_E = 128
CN = CHUNK_E // K
NCH = NPW * K // CHUNK_E

ROWS = 256
NT = N // ROWS


def _graph_body(pos_ref, post_ref, idx_ref):
    b = pl.program_id(0)
    t = pl.program_id(1)
    d2 = jnp.zeros((ROWS, N), jnp.float32)
    for c in range(3):
        col = pos_ref[0, :, c:c + 1]
        row = post_ref[0, c:c + 1, :]
        diff = col - row
        d2 = d2 + diff * diff
    rowi = t * ROWS + lax.broadcasted_iota(jnp.int32, (ROWS, N), 0)
    coli = lax.broadcasted_iota(jnp.int32, (ROWS, N), 1)
    d2 = jnp.where((rowi == coli) | (d2 > R2), jnp.inf, d2)
    base = b * N
    for k in range(K):
        m = jnp.min(d2, axis=1, keepdims=True)
        am = jnp.min(jnp.where(d2 == m, coli, N), axis=1, keepdims=True)
        valid = m != jnp.inf
        idx_ref[:, k:k + 1] = jnp.where(valid, am + base, PAD)
        d2 = jnp.where(coli == am, jnp.inf, d2)


def _graph(pos, post):
    return pl.pallas_call(
        _graph_body,
        grid=(B, NT),
        in_specs=[
            pl.BlockSpec((1, ROWS, 3), lambda b, t: (b, t, 0)),
            pl.BlockSpec((1, 3, N), lambda b, t: (b, 0, 0)),
        ],
        out_specs=pl.BlockSpec((ROWS, K), lambda b, t: (b * NT + t, 0)),
        out_shape=jax.ShapeDtypeStruct((BN, K), jnp.int32),
    )(pos, post)


def _mm_body(x_ref, w_ref, b_ref, o_ref):
    o_ref[...] = (
        jnp.dot(x_ref[...], w_ref[...], preferred_element_type=jnp.float32)
        + b_ref[...]
    )


def _matmul(x, w, bias):
    m, kdim = x.shape
    tile = 2048
    return pl.pallas_call(
        _mm_body,
        grid=(m // tile,),
        in_specs=[
            pl.BlockSpec((tile, kdim), lambda i: (i, 0)),
            pl.BlockSpec((kdim, 128), lambda i: (0, 0)),
            pl.BlockSpec((1, 128), lambda i: (0, 0)),
        ],
        out_specs=pl.BlockSpec((tile, 128), lambda i: (i, 0)),
        out_shape=jax.ShapeDtypeStruct((m, 128), jnp.float32),
    )(x, w, bias)


def _scmax_body(p_hbm, idx_hbm, q_hbm, out_hbm, idx_v, rows_v, q_v, o_v, sem):
    wid = lax.axis_index("s") * NC + lax.axis_index("c")
    nbase = wid * NPW
    pltpu.sync_copy(idx_hbm.at[pl.ds(wid * NCH, NCH)], idx_v)
    pltpu.sync_copy(q_hbm.at[pl.ds(nbase, NPW)], q_v)

    def body(j, carry):
        pltpu.async_copy(p_hbm.at[idx_v.at[j]], rows_v, sem).wait()
        for n in range(CN):
            node = j * CN + n
            for c in range(D // L):
                sl = pl.ds(c * L, L)
                acc = rows_v[n * K, sl]
                for r in range(1, K):
                    acc = jnp.maximum(acc, rows_v[n * K + r, sl])
                o_v[node, sl] = jnp.maximum(acc + q_v[node, sl], 0.0)
        return carry

    lax.fori_loop(0, NCH, body, 0)
    pltpu.sync_copy(o_v, out_hbm.at[pl.ds(nbase, NPW)])


@functools.lru_cache(maxsize=1)
def _scmax_call():
    return functools.partial(
        pl.kernel,
        out_type=jax.ShapeDtypeStruct((BN, D), jnp.float32),
        mesh=plsc.VectorSubcoreMesh(core_axis_name="c", subcore_axis_name="s",
                                    num_cores=NC, num_subcores=NS),
        scratch_types=[
            pltpu.VMEM((NCH, CHUNK_E), jnp.int32),
            pltpu.VMEM((CHUNK_E, D), jnp.float32),
            pltpu.VMEM((NPW, D), jnp.float32),
            pltpu.VMEM((NPW, D), jnp.float32),
            pltpu.SemaphoreType.DMA,
        ],
        compiler_params=pltpu.CompilerParams(use_tc_tiling_on_sc=False),
    )(_scmax_body)


def _scmax(p_pad, idx2, q):
    return _scmax_call()(p_pad, idx2, q)


def kernel(rois, pooled_features, fc1_w, fc1_b, fc2_w, fc2_b):
    b, n, c = pooled_features.shape
    gpf = jnp.concatenate([pooled_features, rois], axis=-1).reshape(BN, c + 7)
    pos = rois[..., :3]
    post = jnp.transpose(pos, (0, 2, 1))
    idx = _graph(pos, post)
    idx2 = idx.reshape(BN * K // CHUNK_E, CHUNK_E)

    w1a, w1b = fc1_w[:, : c + 7], fc1_w[:, c + 7:]
    wc1 = jnp.concatenate([w1a.T, (w1b - w1a).T], axis=1)
    bc1 = jnp.concatenate([jnp.zeros((D,), jnp.float32), fc1_b]).reshape(1, 128)
    pq1 = _matmul(gpf, wc1, bc1)
    neg = jnp.full((16, D), -jnp.inf, jnp.float32)
    x1 = _scmax(jnp.concatenate([pq1[:, :D], neg], axis=0), idx2, pq1[:, D:])

    w2a, w2b = fc2_w[:, :D], fc2_w[:, D:]
    wc2 = jnp.concatenate([w2a.T, (w2b - w2a).T], axis=1)
    bc2 = jnp.concatenate([jnp.zeros((D,), jnp.float32), fc2_b]).reshape(1, 128)
    pq2 = _matmul(x1, wc2, bc2)
    x2 = _scmax(jnp.concatenate([pq2[:, :D], neg], axis=0), idx2, pq2[:, D:])

    return jnp.concatenate([gpf, x1, x2], axis=-1)
